# baseline (speedup 1.0000x reference)
import functools

import jax
import jax.numpy as jnp
import numpy as np
from jax import lax
from jax.experimental import pallas as pl
from jax.experimental.pallas import tpu as pltpu
from jax.experimental.pallas import tpu_sc as plsc

_VOXEL = 0.04
_VOL = 48
_NPAR = 100000
_NPAD = 102400
_TILE_PARENTS = _NPAD // 32
_CHUNK = 128
_NCHUNKS = _TILE_PARENTS // _CHUNK
_TC_BLOCK = 4096
_F_BLOCK = 8192
_FI_MAX = 46 * (_VOL * _VOL + _VOL + 1)

_OFF_XYZ = np.array([
    [0, 0, 0], [1, 0, 0], [0, 1, 0], [0, 0, 1],
    [1, 1, 0], [1, 0, 1], [0, 1, 1], [1, 1, 1],
], dtype=np.int64)
_OFF_FLAT = (_OFF_XYZ @ np.array([_VOL * _VOL, _VOL, 1])).tolist()


def _pack_body(feat, coords, wcat, bvec, origin, w2c, packed):
    ft = jnp.transpose(feat[...])
    s = jax.lax.dot_general(wcat[...], ft, (((0,), (0,)), ((), ())),
                            preferred_element_type=jnp.float32)
    s = s + jnp.transpose(bvec[...])
    loc = jnp.clip(ft[22:23, :] * 2.0, -1.0, 1.0)

    ct = jnp.transpose(coords[...])
    xyzf = ct[1:4, :].astype(jnp.float32)
    cb = xyzf * _VOXEL + jnp.transpose(origin[...])
    A = w2c[0]
    R3 = A[:3, :3]
    t3 = A[:3, 3]
    r = jax.lax.dot_general(R3, cb, (((1,), (0,)), ((), ())),
                            preferred_element_type=jnp.float32) + t3[:, None]

    x = ct[1:2, :] // 2
    y = ct[2:3, :] // 2
    z = ct[3:4, :] // 2
    fi = x * (_VOL * _VOL) + y * _VOL + z
    fif = jax.lax.bitcast_convert_type(fi, jnp.float32)

    zero = jnp.zeros_like(loc)
    vals = jnp.concatenate([s, loc, r, fif, zero], axis=0)
    packed[...] = vals.reshape(_TC_BLOCK // 16, 128)


def _tc_pack(feat, coords, wcat, bvec, origin, w2c):
    grid = _NPAD // _TC_BLOCK
    return pl.pallas_call(
        _pack_body,
        grid=(grid,),
        in_specs=[
            pl.BlockSpec((_TC_BLOCK, 24), lambda i: (i, 0)),
            pl.BlockSpec((_TC_BLOCK, 4), lambda i: (i, 0)),
            pl.BlockSpec((24, 2), lambda i: (0, 0)),
            pl.BlockSpec((1, 2), lambda i: (0, 0)),
            pl.BlockSpec((1, 3), lambda i: (0, 0)),
            pl.BlockSpec((1, 4, 4), lambda i: (0, 0, 0)),
        ],
        out_specs=pl.BlockSpec((_TC_BLOCK // 16, 128), lambda i: (i, 0)),
        out_shape=jax.ShapeDtypeStruct((_NPAD // 16, 128), jnp.float32),
    )(feat, coords, wcat, bvec, origin, w2c)


def _sc_kernel(pk_hbm, table_hbm, dl_hbm, out_hbm, table_v,
               pk_a, pk_b, out_a, out_b, dl_s,
               si_a, si_b, so_a, so_b):
    wid = lax.axis_index("s") * 2 + lax.axis_index("c")
    pltpu.sync_copy(table_hbm, table_v)
    pltpu.sync_copy(dl_hbm, dl_s)

    base = wid * _NCHUNKS
    last = 32 * _NCHUNKS - 1

    def issue_in(m, pk, sem):
        blk = m // 32
        sub = m % 32
        for j in range(7):
            pltpu.async_copy(
                pk_hbm.at[pl.ds(blk * 32768 + j * 4096 + sub * 128, 128)],
                pk.at[pl.ds(j * 128, 128)], sem)

    def drain_in(pk, sem):
        for j in range(7):
            pltpu.make_async_copy(
                pk_hbm.at[pl.ds(0, 128)],
                pk.at[pl.ds(j * 128, 128)], sem).wait()

    _PIECE = _CHUNK * 32

    def drain_out(out, sem):
        pltpu.make_async_copy(
            out, out_hbm.at[pl.ds(0, _PIECE)], sem).wait()

    def compute_piece(m, pk, h, out, sem, warm):
        @pl.when(warm)
        def _():
            drain_out(out, sem)

        dsp = [[plsc.load_gather(dl_s, [jnp.full((16,), k * 3 + j, jnp.int32)])
                for j in range(3)] for k in range(8)]
        zeros = jnp.zeros((16,), jnp.float32)
        iota128 = lax.iota(jnp.int32, 16) * 128

        for g4 in range(2):
            g = h * 2 + g4

            def col(j):
                return pk[pl.ds(j * 128 + g * 16, 16)]

            s_t, s_o, lc = col(0), col(1), col(2)
            bx, by, bz = col(3), col(4), col(5)
            fi = plsc.bitcast(col(6), jnp.int32)
            fi = jnp.clip(fi, 0, _FI_MAX)
            orow0 = iota128 + g4 * (16 * 128)

            for k in range(8):
                g32 = plsc.load_gather(table_v, [fi + _OFF_FLAT[k]])
                tsdf = plsc.bitcast(g32 & jnp.int32(-65536), jnp.float32)
                occ = plsc.bitcast(g32 << 16, jnp.float32)
                rbase = orow0 + k * 16

                def sc(j, v):
                    plsc.store_scatter(out, [rbase + j], v)

                sc(0, s_t)
                sc(1, s_o)
                sc(2, lc)
                sc(3, tsdf)
                sc(4, occ)
                sc(5, bx + dsp[k][0])
                sc(6, by + dsp[k][1])
                sc(7, bz + dsp[k][2])
                sc(8, zeros)

        pltpu.async_copy(
            out, out_hbm.at[pl.ds(m * (_CHUNK * 128) + h * _PIECE, _PIECE)],
            sem)

    def compute(m, pk, warm0):
        compute_piece(m, pk, 0, out_a, so_a, warm0)
        compute_piece(m, pk, 1, out_b, so_b, warm0)
        compute_piece(m, pk, 2, out_a, so_a, m >= 0)
        compute_piece(m, pk, 3, out_b, so_b, m >= 0)

    issue_in(base, pk_a, si_a)

    def body(i, carry):
        m0 = base + 2 * i
        m1 = m0 + 1
        issue_in(m1, pk_b, si_b)
        drain_in(pk_a, si_a)
        compute(m0, pk_a, i > 0)
        issue_in(jnp.minimum(m0 + 2, last), pk_a, si_a)
        drain_in(pk_b, si_b)
        compute(m1, pk_b, i + 1 > 0)
        return carry

    lax.fori_loop(0, (_NCHUNKS - 1) // 2, body, 0)

    m_t = base + _NCHUNKS - 1
    drain_in(pk_a, si_a)
    compute(m_t, pk_a, wid + 1 > 0)
    drain_out(out_a, so_a)
    drain_out(out_b, so_b)


def _sc_assemble(pk, table, deltas):
    mesh = plsc.VectorSubcoreMesh(core_axis_name="c", subcore_axis_name="s")
    run = functools.partial(
        pl.kernel,
        mesh=mesh,
        compiler_params=pltpu.CompilerParams(needs_layout_passes=False),
        out_type=jax.ShapeDtypeStruct((_NPAD * 128,), jnp.float32),
        scratch_types=[
            pltpu.VMEM((_VOL * _VOL * _VOL,), jnp.int32),
            pltpu.VMEM((_CHUNK * 7,), jnp.float32),
            pltpu.VMEM((_CHUNK * 7,), jnp.float32),
            pltpu.VMEM((_CHUNK * 32,), jnp.float32),
            pltpu.VMEM((_CHUNK * 32,), jnp.float32),
            pltpu.VMEM((24,), jnp.float32),
            pltpu.SemaphoreType.DMA,
            pltpu.SemaphoreType.DMA,
            pltpu.SemaphoreType.DMA,
            pltpu.SemaphoreType.DMA,
        ],
    )(_sc_kernel)
    return run(pk, table, deltas)


def _fin_body(rows, out):
    x = rows[...]
    pieces = [x[:, 16 * k:16 * k + 9] for k in range(8)]
    stacked = jnp.concatenate([p[:, None, :] for p in pieces], axis=1)
    out[...] = stacked.reshape(_FIN_BLOCK * 8, 9)


_FIN_BLOCK = 1024


def _tc_finalize(flat):
    rows = flat.reshape(_NPAD, 128)
    grid = (_NPAR + _FIN_BLOCK - 1) // _FIN_BLOCK
    return pl.pallas_call(
        _fin_body,
        grid=(grid,),
        in_specs=[pl.BlockSpec((_FIN_BLOCK, 128), lambda i: (i, 0))],
        out_specs=pl.BlockSpec((_FIN_BLOCK * 8, 9), lambda i: (i, 0)),
        out_shape=jax.ShapeDtypeStruct((_NPAR * 8, 9), jnp.float32),
    )(rows)


def kernel(pre_feat, pre_coords, tsdf_vol, occ_vol, W_tsdf, b_tsdf,
           W_occ, b_occ, vol_origin, world_to_cam):
    coords = pre_coords.astype(jnp.int32)
    wcat = jnp.concatenate([W_tsdf, W_occ], axis=1)
    bvec = jnp.concatenate([b_tsdf, b_occ])[None, :]

    offs = jnp.asarray(_OFF_XYZ * 2, jnp.float32) * _VOXEL
    deltas = (offs @ world_to_cam[0, :3, :3].T).reshape(-1)

    t16 = jax.lax.bitcast_convert_type(
        tsdf_vol.reshape(-1).astype(jnp.bfloat16), jnp.uint16)
    o16 = jax.lax.bitcast_convert_type(
        occ_vol.reshape(-1).astype(jnp.bfloat16), jnp.uint16)
    table = jax.lax.bitcast_convert_type(
        (t16.astype(jnp.uint32) << 16) | o16.astype(jnp.uint32), jnp.int32)

    packed = _tc_pack(pre_feat, coords, wcat, bvec, vol_origin, world_to_cam)
    flat = _sc_assemble(packed.reshape(-1), table, deltas)
    return _tc_finalize(flat)


# speedup vs baseline: 2.5936x; 2.5936x over previous
import functools

import jax
import jax.numpy as jnp
import numpy as np
from jax import lax
from jax.experimental import pallas as pl
from jax.experimental.pallas import tpu as pltpu
from jax.experimental.pallas import tpu_sc as plsc

_VOXEL = 0.04
_VOL = 48
_NPAR = 100000
_NPAD = 102400
_TILE_PARENTS = _NPAD // 32
_CHUNK = 128
_NCHUNKS = _TILE_PARENTS // _CHUNK
_TC_BLOCK = 4096
_F_BLOCK = 8192
_FI_MAX = 46 * (_VOL * _VOL + _VOL + 1)

_OFF_XYZ = np.array([
    [0, 0, 0], [1, 0, 0], [0, 1, 0], [0, 0, 1],
    [1, 1, 0], [1, 0, 1], [0, 1, 1], [1, 1, 1],
], dtype=np.int64)
_OFF_FLAT = (_OFF_XYZ @ np.array([_VOL * _VOL, _VOL, 1])).tolist()


def _pack_body(feat, coords, wcat, bvec, origin, w2c, packed):
    ft = jnp.transpose(feat[...])
    s = jax.lax.dot_general(wcat[...], ft, (((0,), (0,)), ((), ())),
                            preferred_element_type=jnp.float32)
    s = s + jnp.transpose(bvec[...])
    loc = jnp.clip(ft[22:23, :] * 2.0, -1.0, 1.0)

    ct = jnp.transpose(coords[...])
    xyzf = ct[1:4, :].astype(jnp.float32)
    cb = xyzf * _VOXEL + jnp.transpose(origin[...])
    A = w2c[0]
    R3 = A[:3, :3]
    t3 = A[:3, 3]
    r = jax.lax.dot_general(R3, cb, (((1,), (0,)), ((), ())),
                            preferred_element_type=jnp.float32) + t3[:, None]

    x = ct[1:2, :] // 2
    y = ct[2:3, :] // 2
    z = ct[3:4, :] // 2
    fi = x * (_VOL * _VOL) + y * _VOL + z
    fif = jax.lax.bitcast_convert_type(fi, jnp.float32)

    zero = jnp.zeros_like(loc)
    vals = jnp.concatenate([s, loc, r, fif, zero], axis=0)
    packed[...] = vals.reshape(_TC_BLOCK // 16, 128)


def _tc_pack(feat, coords, wcat, bvec, origin, w2c):
    grid = _NPAD // _TC_BLOCK
    return pl.pallas_call(
        _pack_body,
        grid=(grid,),
        in_specs=[
            pl.BlockSpec((_TC_BLOCK, 24), lambda i: (i, 0)),
            pl.BlockSpec((_TC_BLOCK, 4), lambda i: (i, 0)),
            pl.BlockSpec((24, 2), lambda i: (0, 0)),
            pl.BlockSpec((1, 2), lambda i: (0, 0)),
            pl.BlockSpec((1, 3), lambda i: (0, 0)),
            pl.BlockSpec((1, 4, 4), lambda i: (0, 0, 0)),
        ],
        out_specs=pl.BlockSpec((_TC_BLOCK // 16, 128), lambda i: (i, 0)),
        out_shape=jax.ShapeDtypeStruct((_NPAD // 16, 128), jnp.float32),
    )(feat, coords, wcat, bvec, origin, w2c)


def _sc_kernel(pk_hbm, table_hbm, dl_hbm,
               o0, o1, o2, o3, o4, o5, o6, o7, o8, table_v,
               pk_a, pk_b, out_a, out_b, dl_s,
               si_a, si_b, so_a, so_b):
    outs = [o0, o1, o2, o3, o4, o5, o6, o7, o8]
    wid = lax.axis_index("s") * 2 + lax.axis_index("c")
    pltpu.sync_copy(table_hbm, table_v)
    pltpu.sync_copy(dl_hbm, dl_s)

    base = wid * _NCHUNKS
    last = 32 * _NCHUNKS - 1

    def issue_in(m, pk, sem):
        blk = m // 32
        sub = m % 32
        for j in range(7):
            pltpu.async_copy(
                pk_hbm.at[pl.ds(blk * 32768 + j * 4096 + sub * 128, 128)],
                pk.at[pl.ds(j * 128, 128)], sem)

    def drain_in(pk, sem):
        for j in range(7):
            pltpu.make_async_copy(
                pk_hbm.at[pl.ds(0, 128)],
                pk.at[pl.ds(j * 128, 128)], sem).wait()

    _PW = 256

    def drain_out(out, sem):
        for j in range(9):
            pltpu.make_async_copy(
                out.at[pl.ds(j * _PW, _PW)],
                outs[j].at[pl.ds(0, _PW)], sem).wait()

    def compute_piece(m, pk, h, out, sem, warm):
        @pl.when(warm)
        def _():
            drain_out(out, sem)

        dsp = [[plsc.load_gather(dl_s, [jnp.full((16,), k * 3 + j, jnp.int32)])
                for j in range(3)] for k in range(8)]
        zeros = jnp.zeros((16,), jnp.float32)
        iota8 = lax.iota(jnp.int32, 16) * 8

        for g4 in range(2):
            g = h * 2 + g4

            def col(j):
                return pk[pl.ds(j * 128 + g * 16, 16)]

            s_t, s_o, lc = col(0), col(1), col(2)
            bx, by, bz = col(3), col(4), col(5)
            fi = plsc.bitcast(col(6), jnp.int32)
            fi = jnp.clip(fi, 0, _FI_MAX)

            for k in range(8):
                g32 = plsc.load_gather(table_v, [fi + _OFF_FLAT[k]])
                tsdf = plsc.bitcast(g32 & jnp.int32(-65536), jnp.float32)
                occ = plsc.bitcast(g32 << 16, jnp.float32)
                rbase = iota8 + (g4 * 128 + k)

                def sc(j, v):
                    plsc.store_scatter(out, [rbase + j * _PW], v)

                sc(0, s_t)
                sc(1, s_o)
                sc(2, lc)
                sc(3, tsdf)
                sc(4, occ)
                sc(5, bx + dsp[k][0])
                sc(6, by + dsp[k][1])
                sc(7, bz + dsp[k][2])
                sc(8, zeros)

        pid = m * 4 + h
        off = jnp.where(pid < _NPAR * 8 // _PW, pid * _PW, _NPAR * 8)
        for j in range(9):
            pltpu.async_copy(out.at[pl.ds(j * _PW, _PW)],
                             outs[j].at[pl.ds(off, _PW)], sem)

    def compute(m, pk, warm0):
        compute_piece(m, pk, 0, out_a, so_a, warm0)
        compute_piece(m, pk, 1, out_b, so_b, warm0)
        compute_piece(m, pk, 2, out_a, so_a, m >= 0)
        compute_piece(m, pk, 3, out_b, so_b, m >= 0)

    issue_in(base, pk_a, si_a)

    def body(i, carry):
        m0 = base + 2 * i
        m1 = m0 + 1
        issue_in(m1, pk_b, si_b)
        drain_in(pk_a, si_a)
        compute(m0, pk_a, i > 0)
        issue_in(jnp.minimum(m0 + 2, last), pk_a, si_a)
        drain_in(pk_b, si_b)
        compute(m1, pk_b, i + 1 > 0)
        return carry

    lax.fori_loop(0, (_NCHUNKS - 1) // 2, body, 0)

    m_t = base + _NCHUNKS - 1
    drain_in(pk_a, si_a)
    compute(m_t, pk_a, wid + 1 > 0)
    drain_out(out_a, so_a)
    drain_out(out_b, so_b)


def _sc_assemble(pk, table, deltas):
    mesh = plsc.VectorSubcoreMesh(core_axis_name="c", subcore_axis_name="s")
    run = functools.partial(
        pl.kernel,
        mesh=mesh,
        compiler_params=pltpu.CompilerParams(needs_layout_passes=False),
        out_type=[jax.ShapeDtypeStruct((_NPAR * 8 + 256,), jnp.float32)
                  for _ in range(9)],
        scratch_types=[
            pltpu.VMEM((_VOL * _VOL * _VOL,), jnp.int32),
            pltpu.VMEM((_CHUNK * 7,), jnp.float32),
            pltpu.VMEM((_CHUNK * 7,), jnp.float32),
            pltpu.VMEM((9 * 256,), jnp.float32),
            pltpu.VMEM((9 * 256,), jnp.float32),
            pltpu.VMEM((24,), jnp.float32),
            pltpu.SemaphoreType.DMA,
            pltpu.SemaphoreType.DMA,
            pltpu.SemaphoreType.DMA,
            pltpu.SemaphoreType.DMA,
        ],
    )(_sc_kernel)
    return run(pk, table, deltas)


def _tc_finalize(planes):
    n = _NPAR * 8
    return jnp.concatenate([p[:n, None] for p in planes], axis=1)


def kernel(pre_feat, pre_coords, tsdf_vol, occ_vol, W_tsdf, b_tsdf,
           W_occ, b_occ, vol_origin, world_to_cam):
    coords = pre_coords.astype(jnp.int32)
    wcat = jnp.concatenate([W_tsdf, W_occ], axis=1)
    bvec = jnp.concatenate([b_tsdf, b_occ])[None, :]

    offs = jnp.asarray(_OFF_XYZ * 2, jnp.float32) * _VOXEL
    deltas = (offs @ world_to_cam[0, :3, :3].T).reshape(-1)

    t16 = jax.lax.bitcast_convert_type(
        tsdf_vol.reshape(-1).astype(jnp.bfloat16), jnp.uint16)
    o16 = jax.lax.bitcast_convert_type(
        occ_vol.reshape(-1).astype(jnp.bfloat16), jnp.uint16)
    table = jax.lax.bitcast_convert_type(
        (t16.astype(jnp.uint32) << 16) | o16.astype(jnp.uint32), jnp.int32)

    packed = _tc_pack(pre_feat, coords, wcat, bvec, vol_origin, world_to_cam)
    planes = _sc_assemble(packed.reshape(-1), table, deltas)
    return _tc_finalize(planes)
